# Initial kernel scaffold; baseline (speedup 1.0000x reference)
import jax
import jax.numpy as jnp
from jax.experimental import pallas as pl


def kernel(x, batch_map, W1, b1, W2, b2):
    raise NotImplementedError("write your pallas kernel here")


# speedup vs baseline: 5.3477x; 5.3477x over previous
import jax
import jax.numpy as jnp
from jax import lax
from jax.experimental import pallas as pl
from jax.experimental.pallas import tpu as pltpu

N = 160000
B = 1024
D = 256
H = 128
R = 1600
NB = N // R


def _body(bm_ref, x_ref, w1_ref, b1_ref, w2_ref, b2_ref, out_ref,
          num_acc, den_acc):
    g = pl.program_id(0)

    @pl.when(g == 0)
    def _init():
        num_acc[...] = jnp.zeros_like(num_acc)
        den_acc[...] = jnp.zeros_like(den_acc)

    x = x_ref[...]
    h = jnp.tanh(
        jax.lax.dot_general(x, w1_ref[...], (((1,), (0,)), ((), ())),
                            preferred_element_type=jnp.float32)
        + b1_ref[...])
    s = jnp.sum(h * w2_ref[...], axis=1, keepdims=True) + b2_ref[...]
    s = jnp.clip(s, -10.0, 10.0)
    e = jnp.exp(s)

    ids = bm_ref[0, 0, :]
    cols = lax.broadcasted_iota(jnp.int32, (R, B), 1)
    p = (ids[:, None] == cols)
    p_f32 = p.astype(jnp.float32)
    p_bf = p.astype(jnp.bfloat16)

    y = (x * e).astype(jnp.bfloat16)
    num_acc[...] += jax.lax.dot_general(
        p_bf, y, (((0,), (0,)), ((), ())),
        preferred_element_type=jnp.float32)
    den_acc[...] += jax.lax.dot_general(
        p_f32, e, (((0,), (0,)), ((), ())),
        preferred_element_type=jnp.float32)

    @pl.when(g == NB - 1)
    def _finish():
        den = den_acc[...]
        out_ref[...] = jnp.where(den > 0.0, num_acc[...] / den, 0.0)


def kernel(x, batch_map, W1, b1, W2, b2):
    bm3 = batch_map.reshape(NB, 1, R)
    b1r = b1.reshape(1, H)
    w2r = W2.reshape(1, H)
    b2r = b2.reshape(1, 1)
    out = pl.pallas_call(
        _body,
        grid=(NB,),
        in_specs=[
            pl.BlockSpec((1, 1, R), lambda g: (g, 0, 0)),
            pl.BlockSpec((R, D), lambda g: (g, 0)),
            pl.BlockSpec((D, H), lambda g: (0, 0)),
            pl.BlockSpec((1, H), lambda g: (0, 0)),
            pl.BlockSpec((1, H), lambda g: (0, 0)),
            pl.BlockSpec((1, 1), lambda g: (0, 0)),
        ],
        out_specs=pl.BlockSpec((B, D), lambda g: (0, 0)),
        out_shape=jax.ShapeDtypeStruct((B, D), jnp.float32),
        scratch_shapes=[
            pltpu.VMEM((B, D), jnp.float32),
            pltpu.VMEM((B, 1), jnp.float32),
        ],
    )(bm3, x, W1, b1r, w2r, b2r)
    return out
